# Initial kernel scaffold; baseline (speedup 1.0000x reference)
import jax
import jax.numpy as jnp
from jax.experimental import pallas as pl


def kernel(x1, x2, W):
    raise NotImplementedError("write your pallas kernel here")


# speedup vs baseline: 3.3131x; 3.3131x over previous
import functools
import jax
import jax.numpy as jnp
from jax.experimental import pallas as pl
from jax.experimental.pallas import tpu as pltpu

_C = 768
_CH = 384
_KO = 128
_HW = 4096
_INV_HW = 1.0 / 4096.0


def _mean_body(x1_ref, x2_ref, p1_ref, p2_ref):
    p1_ref[...] = jnp.sum(x1_ref[...], axis=2, keepdims=True).transpose(0, 2, 1) * _INV_HW
    p2_ref[...] = jnp.sum(x2_ref[...], axis=2, keepdims=True).transpose(0, 2, 1) * _INV_HW


def _build_m_body(pooled_ref, w_ref, m_ref):
    v = pooled_ref[0, 0, :]
    vj = v[:, None]
    vc = v[None, :]
    ij = jax.lax.broadcasted_iota(jnp.int32, (_C, _C), 0)
    ic = jax.lax.broadcasted_iota(jnp.int32, (_C, _C), 1)
    beats = (vj > vc) | ((vj == vc) & (ij < ic))
    rank = jnp.sum(beats.astype(jnp.int32), axis=0)

    pr = jax.lax.broadcasted_iota(jnp.int32, (_CH, _C), 0)
    top = (rank[None, :] == pr).astype(jnp.float32)
    sel = (rank[None, :] - _CH == pr).astype(jnp.float32)
    wperm = jnp.dot(w_ref[...], sel,
                    preferred_element_type=jnp.float32)
    m_ref[0, :, :] = jnp.concatenate([top, wperm], axis=0)


def _matmul_body(m_ref, x1_ref, x2_ref, out_ref):
    m = m_ref[0]
    out_ref[0, :, :] = (
        jnp.dot(m[:, :_CH], x1_ref[0], preferred_element_type=jnp.float32)
        + jnp.dot(m[:, _CH:], x2_ref[0], preferred_element_type=jnp.float32)
    )


def kernel(x1, x2, W):
    b = x1.shape[0]
    x1f = x1.reshape(b, _CH, _HW)
    x2f = x2.reshape(b, _CH, _HW)

    cblk = 128
    p1, p2 = pl.pallas_call(
        _mean_body,
        grid=(b, _CH // cblk),
        in_specs=[
            pl.BlockSpec((1, cblk, _HW), lambda i, j: (i, j, 0)),
            pl.BlockSpec((1, cblk, _HW), lambda i, j: (i, j, 0)),
        ],
        out_specs=[
            pl.BlockSpec((1, 1, cblk), lambda i, j: (i, 0, j)),
            pl.BlockSpec((1, 1, cblk), lambda i, j: (i, 0, j)),
        ],
        out_shape=[
            jax.ShapeDtypeStruct((b, 1, _CH), jnp.float32),
            jax.ShapeDtypeStruct((b, 1, _CH), jnp.float32),
        ],
    )(x1f, x2f)
    pooled = jnp.concatenate([p1, p2], axis=2)

    m = pl.pallas_call(
        _build_m_body,
        grid=(b,),
        in_specs=[
            pl.BlockSpec((1, 1, _C), lambda i: (i, 0, 0)),
            pl.BlockSpec((_KO, _CH), lambda i: (0, 0)),
        ],
        out_specs=pl.BlockSpec((1, _CH + _KO, _C), lambda i: (i, 0, 0)),
        out_shape=jax.ShapeDtypeStruct((b, _CH + _KO, _C), jnp.float32),
    )(pooled, W)

    hwblk = 1024
    out = pl.pallas_call(
        _matmul_body,
        grid=(b, _HW // hwblk),
        in_specs=[
            pl.BlockSpec((1, _CH + _KO, _C), lambda i, j: (i, 0, 0)),
            pl.BlockSpec((1, _CH, hwblk), lambda i, j: (i, 0, j)),
            pl.BlockSpec((1, _CH, hwblk), lambda i, j: (i, 0, j)),
        ],
        out_specs=pl.BlockSpec((1, _CH + _KO, hwblk), lambda i, j: (i, 0, j)),
        out_shape=jax.ShapeDtypeStruct((b, _CH + _KO, _HW), jnp.float32),
    )(m, x1f, x2f)

    return out.reshape(b, _CH + _KO, 64, 64)


# speedup vs baseline: 3.5528x; 1.0723x over previous
import functools
import jax
import jax.numpy as jnp
from jax.experimental import pallas as pl
from jax.experimental.pallas import tpu as pltpu

_C = 768
_CH = 384
_KO = 128
_HW = 4096
_INV_HW = 1.0 / 4096.0


def _mean_body(x1_ref, x2_ref, p1_ref, p2_ref):
    p1_ref[...] = jnp.sum(x1_ref[...], axis=2, keepdims=True).transpose(0, 2, 1) * _INV_HW
    p2_ref[...] = jnp.sum(x2_ref[...], axis=2, keepdims=True).transpose(0, 2, 1) * _INV_HW


def _fused_body(p1_ref, p2_ref, w_ref, x1_ref, x2_ref, out_ref, m_ref):
    j = pl.program_id(1)

    @pl.when(j == 0)
    def _build_m():
        v = jnp.concatenate([p1_ref[0, 0, :], p2_ref[0, 0, :]])
        vj = v[:, None]
        vc = v[None, :]
        ij = jax.lax.broadcasted_iota(jnp.int32, (_C, _C), 0)
        ic = jax.lax.broadcasted_iota(jnp.int32, (_C, _C), 1)
        beats = (vj > vc) | ((vj == vc) & (ij < ic))
        rank = jnp.sum(beats.astype(jnp.int32), axis=0)
        pr = jax.lax.broadcasted_iota(jnp.int32, (_CH, _C), 0)
        top = (rank[None, :] == pr).astype(jnp.float32)
        sel = (rank[None, :] - _CH == pr).astype(jnp.float32)
        wperm = jnp.dot(w_ref[...], sel,
                        preferred_element_type=jnp.float32)
        m_ref[:_CH, :] = top
        m_ref[_CH:, :] = wperm

    m = m_ref[...]
    out_ref[0, :, :] = (
        jnp.dot(m[:, :_CH], x1_ref[0], preferred_element_type=jnp.float32)
        + jnp.dot(m[:, _CH:], x2_ref[0], preferred_element_type=jnp.float32)
    )


def kernel(x1, x2, W):
    b = x1.shape[0]
    x1f = x1.reshape(b, _CH, _HW)
    x2f = x2.reshape(b, _CH, _HW)

    p1, p2 = pl.pallas_call(
        _mean_body,
        grid=(b,),
        in_specs=[
            pl.BlockSpec((1, _CH, _HW), lambda i: (i, 0, 0)),
            pl.BlockSpec((1, _CH, _HW), lambda i: (i, 0, 0)),
        ],
        out_specs=[
            pl.BlockSpec((1, 1, _CH), lambda i: (i, 0, 0)),
            pl.BlockSpec((1, 1, _CH), lambda i: (i, 0, 0)),
        ],
        out_shape=[
            jax.ShapeDtypeStruct((b, 1, _CH), jnp.float32),
            jax.ShapeDtypeStruct((b, 1, _CH), jnp.float32),
        ],
    )(x1f, x2f)

    hwblk = 2048
    out = pl.pallas_call(
        _fused_body,
        grid=(b, _HW // hwblk),
        in_specs=[
            pl.BlockSpec((1, 1, _CH), lambda i, j: (i, 0, 0)),
            pl.BlockSpec((1, 1, _CH), lambda i, j: (i, 0, 0)),
            pl.BlockSpec((_KO, _CH), lambda i, j: (0, 0)),
            pl.BlockSpec((1, _CH, hwblk), lambda i, j: (i, 0, j)),
            pl.BlockSpec((1, _CH, hwblk), lambda i, j: (i, 0, j)),
        ],
        out_specs=pl.BlockSpec((1, _CH + _KO, hwblk), lambda i, j: (i, 0, j)),
        out_shape=jax.ShapeDtypeStruct((b, _CH + _KO, _HW), jnp.float32),
        scratch_shapes=[pltpu.VMEM((_CH + _KO, _C), jnp.float32)],
    )(p1, p2, W, x1f, x2f)

    return out.reshape(b, _CH + _KO, 64, 64)
